# baseline (speedup 1.0000x reference)
import functools

import jax
import jax.numpy as jnp
from jax import lax
from jax.experimental import pallas as pl
from jax.experimental.pallas import tpu as pltpu
from jax.experimental.pallas import tpu_sc as plsc

_LST = 64
_NB = 9


def _bisect_tau(z, axis):
    d = float(z.shape[axis])
    hi = jnp.max(z, axis=axis, keepdims=True)
    lo = hi - 1.0
    for _ in range(_NB):
        mid = 0.5 * (lo + hi)
        g = jnp.sum(jnp.maximum(z, mid), axis=axis, keepdims=True)
        ge = g >= 1.0 + d * mid
        lo = jnp.where(ge, mid, lo)
        hi = jnp.where(ge, hi, mid)
    sup = (z > lo).astype(jnp.float32)
    c = jnp.sum(sup, axis=axis, keepdims=True)
    s = jnp.sum(z * sup, axis=axis, keepdims=True)
    return jnp.clip((s - 1.0) / c, lo, hi)


def _body(x_ref, o_ref, *, t_block):
    z = x_ref[0]
    n_insts = z.shape[0]
    tau_i = _bisect_tau(z, axis=0)
    pi = jnp.maximum(z - tau_i, 0.0)
    nf = t_block // _LST
    zt = z.T.reshape(nf, _LST, n_insts)
    tau_t = _bisect_tau(zt, axis=1)
    pt = jnp.maximum(zt - tau_t, 0.0).reshape(t_block, n_insts).T
    o_ref[0] = pi * pt


def _tc_call(x):
    batch, n_insts, time = x.shape
    t_block = 2048
    if time % t_block:
        t_block = _LST
    grid = (batch, time // t_block)
    spec = pl.BlockSpec((1, n_insts, t_block), lambda b, t: (b, 0, t))
    return pl.pallas_call(
        functools.partial(_body, t_block=t_block),
        grid=grid,
        in_specs=[spec],
        out_specs=spec,
        out_shape=jax.ShapeDtypeStruct(x.shape, x.dtype),
    )(x)


_TC_SC = 256
_NFC = _TC_SC // _LST
_NW = 32


def _sc_bisect(load, n, interval):
    lo, hi = interval

    def bis(_, lh):
        lo, hi = lh
        mid = 0.5 * (lo + hi)

        def gs(i, acc):
            return acc + jnp.maximum(load(i), mid)

        g = lax.fori_loop(0, n, gs, jnp.zeros((16,), jnp.float32))
        ge = g >= 1.0 + float(n) * mid
        return jnp.where(ge, mid, lo), jnp.where(ge, hi, mid)

    lo, hi = lax.fori_loop(0, _NB, bis, (lo, hi))

    def cs(i, acc):
        c, s = acc
        v = load(i)
        sup = v > lo
        return (c + jnp.where(sup, 1.0, 0.0), s + jnp.where(sup, v, 0.0))

    zz = jnp.zeros((16,), jnp.float32)
    c, s = lax.fori_loop(0, n, cs, (zz, zz))
    return jnp.clip((s - 1.0) / c, lo, hi)


def _sc_maxv(load, n):
    def mx(i, m):
        return jnp.maximum(m, load(i))

    return lax.fori_loop(0, n, mx, jnp.full((16,), -3.4e38, jnp.float32))


def _sc_grid_kernel(body, out_struct, scratch):
    mesh = plsc.VectorSubcoreMesh(
        core_axis_name="c", subcore_axis_name="s", num_cores=2, num_subcores=16
    )
    return pl.kernel(body, out_type=out_struct, mesh=mesh, scratch_types=scratch)


def _sc_wid():
    return lax.axis_index("s") * 2 + lax.axis_index("c")


def _sc_call(x):
    B, NI, T = x.shape
    nfr = T // _LST
    ncols = NI * nfr
    cpt = ncols // _NW
    xt = x.reshape(B, NI, nfr, _LST).transpose(0, 3, 1, 2).reshape(B, _LST, ncols)

    def taut_body(xt_hbm, taut_hbm, xv, tv):
        wid = _sc_wid()
        for b in range(B):
            c0 = wid * cpt
            pltpu.sync_copy(xt_hbm.at[b, :, pl.ds(c0, cpt)], xv)

            def cg(k, _):
                q0 = k * 16
                ld = lambda p: xv[p, pl.ds(q0, 16)]
                m = _sc_maxv(ld, _LST)
                tv[pl.ds(q0, 16)] = _sc_bisect(ld, _LST, (m - 1.0, m))
                return 0

            lax.fori_loop(0, cpt // 16, cg, 0)
            pltpu.sync_copy(tv, taut_hbm.at[b, pl.ds(c0, cpt)])

    taut = _sc_grid_kernel(
        taut_body,
        jax.ShapeDtypeStruct((B, ncols), jnp.float32),
        [pltpu.VMEM((_LST, cpt), jnp.float32), pltpu.VMEM((cpt,), jnp.float32)],
    )(xt)
    taut_full = jnp.broadcast_to(
        taut.reshape(B, NI, nfr)[..., None], (B, NI, nfr, _LST)
    ).reshape(B, NI, T)

    tpt = T // _NW

    def taui_body(x_hbm, taui_hbm, xv, tv):
        wid = _sc_wid()
        for b in range(B):
            t0 = wid * tpt
            pltpu.sync_copy(x_hbm.at[b, :, pl.ds(t0, tpt)], xv)

            def cg(k, _):
                q0 = k * 16
                ld = lambda i: xv[i, pl.ds(q0, 16)]
                m = _sc_maxv(ld, NI)
                tv[pl.ds(q0, 16)] = _sc_bisect(ld, NI, (m - 1.0, m))
                return 0

            lax.fori_loop(0, tpt // 16, cg, 0)
            pltpu.sync_copy(tv, taui_hbm.at[b, pl.ds(t0, tpt)])

    taui = _sc_grid_kernel(
        taui_body,
        jax.ShapeDtypeStruct((B, T), jnp.float32),
        [pltpu.VMEM((NI, tpt), jnp.float32), pltpu.VMEM((tpt,), jnp.float32)],
    )(x)

    def comb_body(x_hbm, tt_hbm, ti_hbm, o_hbm, xv, ttv, tiv, ov):
        wid = _sc_wid()
        for b in range(B):
            t0 = wid * tpt
            pltpu.sync_copy(x_hbm.at[b, :, pl.ds(t0, tpt)], xv)
            pltpu.sync_copy(tt_hbm.at[b, :, pl.ds(t0, tpt)], ttv)
            pltpu.sync_copy(ti_hbm.at[b, pl.ds(t0, tpt)], tiv)

            def rw(i, _):
                def cw(k, _):
                    q0 = k * 16
                    v = xv[i, pl.ds(q0, 16)]
                    ti = tiv[pl.ds(q0, 16)]
                    tt = ttv[i, pl.ds(q0, 16)]
                    ov[i, pl.ds(q0, 16)] = (
                        jnp.maximum(v - ti, 0.0) * jnp.maximum(v - tt, 0.0)
                    )
                    return 0

                lax.fori_loop(0, tpt // 16, cw, 0)
                return 0

            lax.fori_loop(0, NI, rw, 0)
            pltpu.sync_copy(ov, o_hbm.at[b, :, pl.ds(t0, tpt)])

    return _sc_grid_kernel(
        comb_body,
        jax.ShapeDtypeStruct((B, NI, T), jnp.float32),
        [
            pltpu.VMEM((NI, tpt), jnp.float32),
            pltpu.VMEM((NI, tpt), jnp.float32),
            pltpu.VMEM((tpt,), jnp.float32),
            pltpu.VMEM((NI, tpt), jnp.float32),
        ],
    )(x, taut_full, taui)


_B_SC = 1


def kernel(midis_out):
    if _B_SC:
        out_sc = _sc_call(midis_out[:_B_SC])
        out_tc = _tc_call(midis_out[_B_SC:])
        return jnp.concatenate([out_sc, out_tc], axis=0)
    return _tc_call(midis_out)


# speedup vs baseline: 2.6583x; 2.6583x over previous
import functools

import jax
import jax.numpy as jnp
from jax import lax
from jax.experimental import pallas as pl
from jax.experimental.pallas import tpu as pltpu
from jax.experimental.pallas import tpu_sc as plsc

_LST = 64
_NB = 8


def _bisect_tau(z, axis):
    d = float(z.shape[axis])
    hi = jnp.max(z, axis=axis, keepdims=True)
    lo = hi - 1.0
    for _ in range(_NB):
        mid = 0.5 * (lo + hi)
        g = jnp.sum(jnp.maximum(z, mid), axis=axis, keepdims=True)
        ge = g >= 1.0 + d * mid
        lo = jnp.where(ge, mid, lo)
        hi = jnp.where(ge, hi, mid)
    sup = (z > lo).astype(jnp.float32)
    c = jnp.sum(sup, axis=axis, keepdims=True)
    s = jnp.sum(z * sup, axis=axis, keepdims=True)
    return jnp.clip((s - 1.0) / c, lo, hi)


def _body(x_ref, o_ref, *, t_block):
    z = x_ref[0]
    n_insts = z.shape[0]
    tau_i = _bisect_tau(z, axis=0)
    pi = jnp.maximum(z - tau_i, 0.0)
    nf = t_block // _LST
    zt = z.T.reshape(nf, _LST, n_insts)
    tau_t = _bisect_tau(zt, axis=1)
    pt = jnp.maximum(zt - tau_t, 0.0).reshape(t_block, n_insts).T
    o_ref[0] = pi * pt


def _tc_call(x):
    batch, n_insts, time = x.shape
    t_block = 2048
    if time % t_block:
        t_block = _LST
    grid = (batch, time // t_block)
    spec = pl.BlockSpec((1, n_insts, t_block), lambda b, t: (b, 0, t))
    return pl.pallas_call(
        functools.partial(_body, t_block=t_block),
        grid=grid,
        in_specs=[spec],
        out_specs=spec,
        out_shape=jax.ShapeDtypeStruct(x.shape, x.dtype),
    )(x)


def kernel(midis_out):
    return _tc_call(midis_out)


# speedup vs baseline: 2.7915x; 1.0501x over previous
import functools

import jax
import jax.numpy as jnp
from jax import lax
from jax.experimental import pallas as pl
from jax.experimental.pallas import tpu as pltpu
from jax.experimental.pallas import tpu_sc as plsc

_LST = 64
_NB = 8


def _bisect_tau(z, axis):
    d = float(z.shape[axis])
    hi = jnp.max(z, axis=axis, keepdims=True)
    lo = hi - 1.0
    for _ in range(_NB):
        mid = 0.5 * (lo + hi)
        g = jnp.sum(jnp.maximum(z, mid), axis=axis, keepdims=True)
        ge = g >= 1.0 + d * mid
        lo = jnp.where(ge, mid, lo)
        hi = jnp.where(ge, hi, mid)
    sup = (z > lo).astype(jnp.float32)
    c = jnp.sum(sup, axis=axis, keepdims=True)
    s = jnp.sum(z * sup, axis=axis, keepdims=True)
    return jnp.clip((s - 1.0) / c, lo, hi)


def _body(x_ref, o_ref, *, t_block):
    z = x_ref[0]
    n_insts = z.shape[0]
    tau_i = _bisect_tau(z, axis=0)
    pi = jnp.maximum(z - tau_i, 0.0)
    nf = t_block // _LST
    zt = z.T.reshape(nf, _LST, n_insts)
    tau_t = _bisect_tau(zt, axis=1)
    pt = jnp.maximum(zt - tau_t, 0.0).reshape(t_block, n_insts).T
    o_ref[0] = pi * pt


def _tc_call(x):
    batch, n_insts, time = x.shape
    t_block = 4096
    if time % t_block:
        t_block = _LST
    grid = (batch, time // t_block)
    spec = pl.BlockSpec((1, n_insts, t_block), lambda b, t: (b, 0, t))
    return pl.pallas_call(
        functools.partial(_body, t_block=t_block),
        grid=grid,
        in_specs=[spec],
        out_specs=spec,
        out_shape=jax.ShapeDtypeStruct(x.shape, x.dtype),
    )(x)


def kernel(midis_out):
    return _tc_call(midis_out)
